# baseline (speedup 1.0000x reference)
import jax
import jax.numpy as jnp
from jax import lax
from jax.experimental import pallas as pl
from jax.experimental.pallas import tpu as pltpu
from jax.experimental.pallas import tpu_sc as plsc

D_MODEL = 64
SEQ = 200
BATCH = 4096
N_TAB = 1000000
NUM_CORES = 2
NUM_SUBCORES = 16
NW = NUM_CORES * NUM_SUBCORES
PCOLS = 128

K1_W = 128
K1_UNITS = N_TAB // K1_W
K1_UPW = -(-K1_UNITS // NW)

K2_UNITS = SEQ * (BATCH // 128)
K2_UPW = K2_UNITS // NW
NBC = BATCH // 128


def _k1_body(tabT, tailT, P, S0, S1, S2, D0, D1,
             si0, si1, si2, so0, so1):
    _LANE = jnp.arange(16, dtype=jnp.int32)
    wid = lax.axis_index("s") * NUM_CORES + lax.axis_index("c")
    lo = wid * K1_UPW
    hi = jnp.minimum(K1_UNITS, lo + K1_UPW)
    S = [S0, S1, S2]
    si = [si0, si1, si2]

    def slab_src(u, dh):
        return tabT.at[pl.ds(8 * dh, 8), pl.ds(u * K1_W, K1_W)]

    def start_slab(u, dh):
        pltpu.async_copy(slab_src(u, dh), S[dh % 3], si[dh % 3])

    def transpose_slab(Sb, D, dh):
        colbase = 8 * dh

        _Z = _LANE * 0

        @plsc.parallel_loop(0, K1_W // 16, unroll=4)
        def _(ig):
            rows = 16 * ig + _LANE
            for d in range(8):
                v = Sb[d, pl.ds(16 * ig, 16)]
                plsc.store_scatter(D, [rows, _Z + (colbase + d)], v)

    def unit(u, D, so):
        start_slab(u, 0)
        start_slab(u, 1)
        start_slab(u, 2)
        for dh in range(8):
            pltpu.make_async_copy(slab_src(u, dh), S[dh % 3], si[dh % 3]).wait()
            transpose_slab(S[dh % 3], D, dh)
            if dh + 3 < 8:
                start_slab(u, dh + 3)
        pltpu.async_copy(D.at[:, pl.ds(0, PCOLS)],
                         P.at[pl.ds(u * K1_W, K1_W), :], so)

    @pl.when(lo < hi)
    def _():
        def body(u, carry):
            even = ((u - lo) % 2) == 0

            @pl.when(even)
            def _():
                @pl.when(u - lo >= 2)
                def _():
                    pltpu.make_async_copy(
                        D0.at[:, pl.ds(0, PCOLS)],
                        P.at[pl.ds((u - 2) * K1_W, K1_W), :], so0).wait()
                unit(u, D0, so0)

            @pl.when(jnp.logical_not(even))
            def _():
                @pl.when(u - lo >= 2)
                def _():
                    pltpu.make_async_copy(
                        D1.at[:, pl.ds(0, PCOLS)],
                        P.at[pl.ds((u - 2) * K1_W, K1_W), :], so1).wait()
                unit(u, D1, so1)
            return carry

        lax.fori_loop(lo, hi, body, 0)

        n = hi - lo
        last_even = ((n - 1) % 2) == 0

        @pl.when(last_even)
        def _():
            pltpu.make_async_copy(
                D0.at[:, pl.ds(0, PCOLS)],
                P.at[pl.ds((hi - 1) * K1_W, K1_W), :], so0).wait()

            @pl.when(n >= 2)
            def _():
                pltpu.make_async_copy(
                    D1.at[:, pl.ds(0, PCOLS)],
                    P.at[pl.ds((hi - 2) * K1_W, K1_W), :], so1).wait()

        @pl.when(jnp.logical_not(last_even))
        def _():
            pltpu.make_async_copy(
                D1.at[:, pl.ds(0, PCOLS)],
                P.at[pl.ds((hi - 1) * K1_W, K1_W), :], so1).wait()

            @pl.when(n >= 2)
            def _():
                pltpu.make_async_copy(
                    D0.at[:, pl.ds(0, PCOLS)],
                    P.at[pl.ds((hi - 2) * K1_W, K1_W), :], so0).wait()

    @pl.when(wid == NW - 1)
    def _():
        pltpu.sync_copy(tailT, D1.at[pl.ds(0, 64), pl.ds(0, 128)])

        @plsc.parallel_loop(0, 8, unroll=2)
        def _(ig):
            rows = 16 * ig + _LANE
            for d0 in range(0, D_MODEL, 8):
                for d in range(8):
                    v = D1[d0 + d, pl.ds(16 * ig, 16)]
                    plsc.store_scatter(D0, [rows, _LANE * 0 + (d0 + d)], v)
        pltpu.sync_copy(D0.at[pl.ds(0, 128), pl.ds(0, PCOLS)],
                        P.at[pl.ds(N_TAB - 128, 128), :])


def _k2_body(P, idxT, out, iv0, iv1, G0, G1, GC0, GC1, O0, O1,
             xi0, xi1, gs0, gs1, gc0, gc1, os0, os1):
    _LANE = jnp.arange(16, dtype=jnp.int32)
    wid = lax.axis_index("s") * NUM_CORES + lax.axis_index("c")
    lo = wid * K2_UPW
    hi = lo + K2_UPW
    iv = [iv0, iv1]
    G = [G0, G1]
    GC = [GC0, GC1]
    O = [O0, O1]
    xi = [xi0, xi1]
    gs = [gs0, gs1]
    gc = [gc0, gc1]
    os_ = [os0, os1]

    def idx_src(u):
        s = u // NBC
        bc = u % NBC
        return idxT.at[s, pl.ds(bc * 128, 128)]

    def start_idx(u, b):
        pltpu.async_copy(idx_src(u), iv[b], xi[b])

    def start_gather(b):
        pltpu.async_copy(P.at[iv[b]], G[b], gs[b])

    def start_compact(b):
        pltpu.async_copy(G[b].at[:, pl.ds(0, D_MODEL)],
                         GC[b].at[:, pl.ds(0, D_MODEL)], gc[b])

    def out_dst(u):
        s = u // NBC
        bc = u % NBC
        return out.at[s, :, pl.ds(bc * 128, 128)]

    def step(u, b):
        pltpu.make_async_copy(P.at[iv[b]], G[b], gs[b]).wait()

        @plsc.parallel_loop(0, 128, unroll=4)
        def _(tok):
            for d0 in range(0, D_MODEL, 16):
                GC[b][tok, pl.ds(d0, 16)] = G[b][tok, pl.ds(d0, 16)]

        @pl.when(u + 2 < hi)
        def _():
            pltpu.make_async_copy(idx_src(u + 2), iv[b], xi[b]).wait()
            start_gather(b)

        @pl.when(u - 2 >= lo)
        def _():
            pltpu.make_async_copy(O[b], out_dst(u - 2), os_[b]).wait()

        _Z = _LANE * 0
        for bg in range(8):
            rows = 16 * bg + _LANE

            @plsc.parallel_loop(0, D_MODEL, unroll=8)
            def _(d):
                v = plsc.load_gather(GC[b], [rows, _Z + d])
                O[b][d, pl.ds(16 * bg, 16)] = v

        pltpu.async_copy(O[b], out_dst(u), os_[b])

        @pl.when(u + 3 < hi)
        def _():
            start_idx(u + 3, 1 - b)

    start_idx(lo, 0)
    start_idx(lo + 1, 1)
    pltpu.make_async_copy(idx_src(lo), iv[0], xi[0]).wait()
    start_gather(0)
    pltpu.make_async_copy(idx_src(lo + 1), iv[1], xi[1]).wait()
    start_gather(1)
    start_idx(lo + 2, 0)

    def body(k2_, carry):
        u0 = lo + 2 * k2_
        step(u0, 0)
        step(u0 + 1, 1)
        return carry

    lax.fori_loop(0, K2_UPW // 2, body, 0)

    pltpu.make_async_copy(O[0], out_dst(hi - 2), os_[0]).wait()
    pltpu.make_async_copy(O[1], out_dst(hi - 1), os_[1]).wait()


def kernel(time_ids, pe_weight):
    mesh = plsc.VectorSubcoreMesh(core_axis_name="c", subcore_axis_name="s")
    tabT = pe_weight.T
    tailT = pe_weight[N_TAB - 128:, :].T
    idxT = time_ids.astype(jnp.int32).T

    k1 = pl.kernel(
        _k1_body,
        out_type=jax.ShapeDtypeStruct((N_TAB, PCOLS), jnp.float32),
        mesh=mesh,
        compiler_params=pltpu.CompilerParams(needs_layout_passes=False),
        scratch_types=[
            pltpu.VMEM((8, K1_W), jnp.float32),
            pltpu.VMEM((8, K1_W), jnp.float32),
            pltpu.VMEM((8, K1_W), jnp.float32),
            pltpu.VMEM((K1_W, PCOLS + 1), jnp.float32),
            pltpu.VMEM((K1_W, PCOLS + 1), jnp.float32),
            pltpu.SemaphoreType.DMA,
            pltpu.SemaphoreType.DMA,
            pltpu.SemaphoreType.DMA,
            pltpu.SemaphoreType.DMA,
            pltpu.SemaphoreType.DMA,
        ],
    )
    P = k1(tabT, tailT)

    k2 = pl.kernel(
        _k2_body,
        out_type=jax.ShapeDtypeStruct((SEQ, D_MODEL, BATCH), jnp.float32),
        mesh=mesh,
        compiler_params=pltpu.CompilerParams(needs_layout_passes=False),
        scratch_types=(
            [pltpu.VMEM((128,), jnp.int32) for _ in range(2)]
            + [pltpu.VMEM((128, PCOLS), jnp.float32) for _ in range(2)]
            + [pltpu.VMEM((128, D_MODEL + 1), jnp.float32) for _ in range(2)]
            + [pltpu.VMEM((D_MODEL, 128), jnp.float32) for _ in range(2)]
            + [pltpu.SemaphoreType.DMA for _ in range(8)]
        ),
    )
    out3 = k2(P, idxT)
    return out3.transpose(2, 0, 1)
